# baseline (speedup 1.0000x reference)
import dataclasses
import functools

import jax
import jax.numpy as jnp
import numpy as np
from jax import lax
from jax.experimental import pallas as pl
from jax.experimental.pallas import tpu as pltpu
from jax.experimental.pallas import tpu_sc as plsc

NC = 2
NS = 16
NW = NC * NS
LANES = 16

N = 10000
NPAD = 10240
D = 128
ROWBLK = NPAD // NS

DCHUNK = 128
CHUNK = 64
T0 = 160
T1 = 160
BH = max(T0, T1) // 2

_P = np.arange(D).reshape(D // 32, 2, 16).transpose(0, 2, 1).reshape(D)

_mesh = plsc.VectorSubcoreMesh(core_axis_name="c", subcore_axis_name="s")

_sc_params = pltpu.CompilerParams()
if "needs_layout_passes" in pltpu.CompilerParams.__dataclass_fields__:
    _sc_params = dataclasses.replace(_sc_params, needs_layout_passes=False)
if "use_tc_tiling_on_sc" in pltpu.CompilerParams.__dataclass_fields__:
    _sc_params = dataclasses.replace(_sc_params, use_tc_tiling_on_sc=False)


def _deg_body(nch, dst_hbm, ew_hbm, out_hbm, idx_v, ew_v, zero_v, acc_sh, dsem):
    cid = lax.axis_index("c")
    sid = lax.axis_index("s")
    wid = cid * NS + sid
    ebase = pl.multiple_of(wid * nch, 8)
    rbase = pl.multiple_of(sid * ROWBLK, 8)

    @pl.loop(0, ROWBLK, step=LANES)
    def _(i):
        zero_v[pl.ds(i, LANES)] = jnp.zeros((LANES,), jnp.float32)

    pltpu.sync_copy(dst_hbm.at[pl.ds(ebase, nch)], idx_v)
    pltpu.sync_copy(ew_hbm.at[pl.ds(ebase, nch)], ew_v)
    pltpu.sync_copy(zero_v, acc_sh.at[pl.ds(rbase, ROWBLK)])
    plsc.subcore_barrier()

    @pl.loop(0, nch)
    def _(j):
        pltpu.async_copy(ew_v.at[j], acc_sh.at[idx_v.at[j]], dsem, add=True)

    @pl.loop(0, nch)
    def _(j):
        pltpu.make_async_copy(ew_v.at[j], acc_sh.at[idx_v.at[j]], dsem).wait()

    plsc.subcore_barrier()
    pltpu.sync_copy(acc_sh.at[pl.ds(rbase, ROWBLK)],
                    out_hbm.at[cid, pl.ds(rbase, ROWBLK)])


def _deg_call(dst2, ew2, nch):
    k = functools.partial(
        pl.kernel,
        out_type=jax.ShapeDtypeStruct((NC, NPAD), jnp.float32),
        mesh=_mesh,
        scratch_types=[
            pltpu.VMEM((nch, DCHUNK), jnp.int32),
            pltpu.VMEM((nch, DCHUNK), jnp.float32),
            pltpu.VMEM((ROWBLK,), jnp.float32),
            pltpu.VMEM_SHARED((NPAD,), jnp.float32),
            pltpu.SemaphoreType.DMA,
        ],
    )(functools.partial(_deg_body, nch))
    return k(dst2, ew2)


def _agg_body(h_hbm, src_hbm, dst_hbm, ew_hbm, out_hbm,
              src_v, dst_v, ew_v, bb0, bb1, bb2, bb3, fb0, fb1, acc_sh,
              gs0, gs1, gs2, gs3, ss0, ss1):
    cid = lax.axis_index("c")
    sid = lax.axis_index("s")
    base = jnp.where(cid == 0, sid * T0, NS * T0 + sid * T1)
    half_n = pl.multiple_of(jnp.where(cid == 0, T0 // 2, T1 // 2), 8)
    base = pl.multiple_of(base, 8)

    bbufs, gsems = [bb0, bb1, bb2, bb3], [gs0, gs1, gs2, gs3]
    fbufs, ssems = [fb0, fb1], [ss0, ss1]

    def start_gather(buf, gsem, cj):
        pltpu.async_copy(h_hbm.at[src_v.at[cj]], buf, gsem)

    def wait_gather(buf, gsem):
        pltpu.make_async_copy(h_hbm.at[src_v.at[0]], buf, gsem).wait()

    def scale(bbuf, fbuf, cj):
        @pl.loop(0, CHUNK, step=4)
        def _(r0):
            for rr in range(4):
                r = r0 + rr
                sv = plsc.load_gather(
                    ew_v, [jnp.full((LANES,), cj, jnp.int32),
                           jnp.full((LANES,), r, jnp.int32)])
                for g in range(D // 32):
                    w = plsc.bitcast(bbuf[r, pl.ds(g * LANES, LANES)],
                                     jnp.bfloat16)
                    a, b = plsc.unpack(w, format=plsc.PackFormat.INTERLEAVED)
                    fbuf[r, pl.ds(g * 32, LANES)] = a * sv
                    fbuf[r, pl.ds(g * 32 + LANES, LANES)] = b * sv

    def start_scat(fbuf, ssem, cj):
        pltpu.async_copy(fbuf, acc_sh.at[dst_v.at[cj]], ssem, add=True)

    def wait_scat(fbuf, ssem):
        pltpu.make_async_copy(fbuf, acc_sh.at[dst_v.at[0]], ssem).wait()

    @pl.loop(0, CHUNK)
    def _(r):
        for jj in range(D // LANES):
            fb0[r, pl.ds(jj * LANES, LANES)] = jnp.zeros((LANES,), jnp.float32)

    for kk in range(ROWBLK // CHUNK):
        off = pl.multiple_of(sid * ROWBLK + kk * CHUNK, 8)
        pltpu.sync_copy(fb0, acc_sh.at[pl.ds(off, CHUNK)])
    plsc.subcore_barrier()

    for half in range(2):
        hb = pl.multiple_of(base + half * half_n, 8)
        pltpu.sync_copy(src_hbm.at[pl.ds(hb, BH)], src_v)
        pltpu.sync_copy(dst_hbm.at[pl.ds(hb, BH)], dst_v)
        pltpu.sync_copy(ew_hbm.at[pl.ds(hb, BH)], ew_v)

        for i in range(4):
            start_gather(bbufs[i], gsems[i], i)
        for i in range(4):
            wait_gather(bbufs[i], gsems[i])
            if i >= 2:
                wait_scat(fbufs[i % 2], ssems[i % 2])
            scale(bbufs[i], fbufs[i % 2], i)
            start_scat(fbufs[i % 2], ssems[i % 2], i)
            start_gather(bbufs[i], gsems[i], jnp.minimum(4 + i, half_n - 1))

        @pl.loop(4, half_n, step=4)
        def _(j):
            for i in range(4):
                wait_gather(bbufs[i], gsems[i])
                wait_scat(fbufs[i % 2], ssems[i % 2])
                scale(bbufs[i], fbufs[i % 2], j + i)
                start_scat(fbufs[i % 2], ssems[i % 2], j + i)
                start_gather(bbufs[i], gsems[i],
                             jnp.minimum(j + i + 4, half_n - 1))

        for i in range(4):
            wait_gather(bbufs[i], gsems[i])
        wait_scat(fb0, ss0)
        wait_scat(fb1, ss1)

    plsc.subcore_barrier()
    for kk in range(ROWBLK // CHUNK):
        off = pl.multiple_of(sid * ROWBLK + kk * CHUNK, 8)
        pltpu.sync_copy(acc_sh.at[pl.ds(off, CHUNK)],
                        out_hbm.at[cid, pl.ds(off, CHUNK)])


def _agg_call(hpb, src2, dst2, ew2):
    k = functools.partial(
        pl.kernel,
        out_type=jax.ShapeDtypeStruct((NC, NPAD, D), jnp.float32),
        mesh=_mesh,
        scratch_types=[
            pltpu.VMEM((BH, CHUNK), jnp.int32),
            pltpu.VMEM((BH, CHUNK), jnp.int32),
            pltpu.VMEM((BH, CHUNK), jnp.float32),
            pltpu.VMEM((CHUNK, D // 2), jnp.int32),
            pltpu.VMEM((CHUNK, D // 2), jnp.int32),
            pltpu.VMEM((CHUNK, D // 2), jnp.int32),
            pltpu.VMEM((CHUNK, D // 2), jnp.int32),
            pltpu.VMEM((CHUNK, D), jnp.float32),
            pltpu.VMEM((CHUNK, D), jnp.float32),
            pltpu.VMEM_SHARED((NPAD, D), jnp.float32),
            pltpu.SemaphoreType.DMA,
            pltpu.SemaphoreType.DMA,
            pltpu.SemaphoreType.DMA,
            pltpu.SemaphoreType.DMA,
            pltpu.SemaphoreType.DMA,
            pltpu.SemaphoreType.DMA,
        ],
        compiler_params=_sc_params,
    )(_agg_body)
    return k(hpb, src2, dst2, ew2)


def _matmul_body(x_ref, w_ref, o_ref):
    o_ref[...] = jnp.dot(x_ref[...], w_ref[...],
                         preferred_element_type=jnp.float32)


def _matmul_call(xp, W):
    return pl.pallas_call(
        _matmul_body,
        out_shape=jax.ShapeDtypeStruct((NPAD, D), jnp.float32),
    )(xp, W)


def _prep_body(part_ref, h_ref, hq_ref, dinvb_ref, hp_ref, hpb_ref):
    deg = part_ref[0] + part_ref[1] + 1.0
    dinv = jnp.where(deg > 0, lax.rsqrt(deg), 0.0)
    dinvb = jnp.broadcast_to(dinv[:, :, None], h_ref.shape)
    dinvb_ref[...] = dinvb
    hp_ref[...] = dinvb * h_ref[...]
    hpb_ref[...] = (dinvb * hq_ref[...]).astype(jnp.bfloat16)


def _prep_call(deg_part3, h1_3, h1q_3):
    r = NPAD // D
    return pl.pallas_call(
        _prep_body,
        out_shape=(jax.ShapeDtypeStruct((r, D, D), jnp.float32),
                   jax.ShapeDtypeStruct((r, D, D), jnp.float32),
                   jax.ShapeDtypeStruct((r, D, D), jnp.bfloat16)),
    )(deg_part3, h1_3, h1q_3)


def _mid_body(part_ref, hp_ref, dinvb_ref, w_ref, wq_ref, b_ref, a_ref,
              o_ref, ob_ref):
    a = a_ref[0, 0]
    z = dinvb_ref[...] * (part_ref[0] + part_ref[1] + hp_ref[...]) + b_ref[...]
    act = jnp.maximum(z, 0.0) + a * jnp.minimum(z, 0.0)
    h2 = jnp.dot(act, w_ref[...], preferred_element_type=jnp.float32)
    h2q = jnp.dot(act, wq_ref[...], preferred_element_type=jnp.float32)
    o_ref[...] = dinvb_ref[...] * h2
    ob_ref[...] = (dinvb_ref[...] * h2q).astype(jnp.bfloat16)


def _mid_call(part1, hp1, dinvb, W2, W2q, b1, prelu_a):
    return pl.pallas_call(
        _mid_body,
        out_shape=(jax.ShapeDtypeStruct((NPAD, D), jnp.float32),
                   jax.ShapeDtypeStruct((NPAD, D), jnp.bfloat16)),
    )(part1, hp1, dinvb, W2, W2q, b1, prelu_a)


def _fin_body(part_ref, hp_ref, dinvb_ref, b_ref, o_ref):
    o_ref[...] = (dinvb_ref[...] * (part_ref[0] + part_ref[1] + hp_ref[...])
                  + b_ref[...])


def _fin_call(part2, hp2, dinvb, b2):
    return pl.pallas_call(
        _fin_body,
        out_shape=jax.ShapeDtypeStruct((NPAD, D), jnp.float32),
    )(part2, hp2, dinvb, b2)


def kernel(x, edge_index, edge_weight, W1, b1, W2, b2, prelu_a):
    n = x.shape[0]
    e = edge_weight.shape[0]

    dept = -(-e // (NW * DCHUNK * 8)) * DCHUNK * 8
    dnch = dept // DCHUNK
    dpad = dept * NW

    assert NS * (T0 + T1) * CHUNK >= e
    apad = (NS * T0 + (NS - 1) * T1 + T1 // 2 + BH) * CHUNK

    pad = max(dpad, apad)
    src_p = jnp.pad(edge_index[0], (0, pad - e))
    dst_p = jnp.pad(edge_index[1], (0, pad - e))
    ew_p = jnp.pad(edge_weight, (0, pad - e))
    xp = jnp.pad(x, ((0, NPAD - n), (0, 0)))

    srcA = src_p[:apad].reshape(apad // CHUNK, CHUNK)
    dstA = dst_p[:apad].reshape(apad // CHUNK, CHUNK)
    ewA = ew_p[:apad].reshape(apad // CHUNK, CHUNK)
    dstD = dst_p[:dpad].reshape(dpad // DCHUNK, DCHUNK)
    ewD = ew_p[:dpad].reshape(dpad // DCHUNK, DCHUNK)

    W1q = W1[:, _P]
    W2q = W2[:, _P]
    deg_part = _deg_call(dstD, ewD, dnch)
    h1 = _matmul_call(xp, W1)
    h1q = _matmul_call(xp, W1q)
    dinvb3, hp1_3, hp1b_3 = _prep_call(
        deg_part.reshape(NC, NPAD // D, D),
        h1.reshape(NPAD // D, D, D),
        h1q.reshape(NPAD // D, D, D))
    dinvb = dinvb3.reshape(NPAD, D)
    hp1 = hp1_3.reshape(NPAD, D)

    def _as_i32(hb):
        return lax.bitcast_convert_type(
            hb.reshape(NPAD, D // 2, 2), jnp.int32)

    hp1b = _as_i32(hp1b_3.reshape(NPAD, D))
    part1 = _agg_call(hp1b, srcA, dstA, ewA)
    hp2, hp2b = _mid_call(part1, hp1, dinvb, W2, W2q,
                          b1.reshape(1, D), prelu_a.reshape(1, 1))
    part2 = _agg_call(_as_i32(hp2b), srcA, dstA, ewA)
    outp = _fin_call(part2, hp2, dinvb, b2.reshape(1, D))
    return outp[:n]


# speedup vs baseline: 1.0634x; 1.0634x over previous
import dataclasses
import functools

import jax
import jax.numpy as jnp
import numpy as np
from jax import lax
from jax.experimental import pallas as pl
from jax.experimental.pallas import tpu as pltpu
from jax.experimental.pallas import tpu_sc as plsc

NC = 2
NS = 16
NW = NC * NS
LANES = 16

N = 10000
NPAD = 10240
D = 128
ROWBLK = NPAD // NS

DCHUNK = 128
CHUNK = 64
T0 = 160
T1 = 160
BH = max(T0, T1) // 2

_P = np.arange(D).reshape(D // 32, 2, 16).transpose(0, 2, 1).reshape(D)

_mesh = plsc.VectorSubcoreMesh(core_axis_name="c", subcore_axis_name="s")

_sc_params = pltpu.CompilerParams()
if "needs_layout_passes" in pltpu.CompilerParams.__dataclass_fields__:
    _sc_params = dataclasses.replace(_sc_params, needs_layout_passes=False)
if "use_tc_tiling_on_sc" in pltpu.CompilerParams.__dataclass_fields__:
    _sc_params = dataclasses.replace(_sc_params, use_tc_tiling_on_sc=False)


def _deg_body(nch, dst_hbm, ew_hbm, out_hbm, idx_v, ew_v, zero_v, acc_sh, dsem):
    cid = lax.axis_index("c")
    sid = lax.axis_index("s")
    wid = cid * NS + sid
    ebase = pl.multiple_of(wid * nch, 8)
    rbase = pl.multiple_of(sid * ROWBLK, 8)

    @pl.loop(0, ROWBLK, step=LANES)
    def _(i):
        zero_v[pl.ds(i, LANES)] = jnp.zeros((LANES,), jnp.float32)

    pltpu.sync_copy(dst_hbm.at[pl.ds(ebase, nch)], idx_v)
    pltpu.sync_copy(ew_hbm.at[pl.ds(ebase, nch)], ew_v)
    pltpu.sync_copy(zero_v, acc_sh.at[pl.ds(rbase, ROWBLK)])
    plsc.subcore_barrier()

    @pl.loop(0, nch)
    def _(j):
        pltpu.async_copy(ew_v.at[j], acc_sh.at[idx_v.at[j]], dsem, add=True)

    @pl.loop(0, nch)
    def _(j):
        pltpu.make_async_copy(ew_v.at[j], acc_sh.at[idx_v.at[j]], dsem).wait()

    plsc.subcore_barrier()
    pltpu.sync_copy(acc_sh.at[pl.ds(rbase, ROWBLK)],
                    out_hbm.at[cid, pl.ds(rbase, ROWBLK)])


def _deg_call(dst2, ew2, nch):
    k = functools.partial(
        pl.kernel,
        out_type=jax.ShapeDtypeStruct((NC, NPAD), jnp.float32),
        mesh=_mesh,
        scratch_types=[
            pltpu.VMEM((nch, DCHUNK), jnp.int32),
            pltpu.VMEM((nch, DCHUNK), jnp.float32),
            pltpu.VMEM((ROWBLK,), jnp.float32),
            pltpu.VMEM_SHARED((NPAD,), jnp.float32),
            pltpu.SemaphoreType.DMA,
        ],
    )(functools.partial(_deg_body, nch))
    return k(dst2, ew2)


def _agg_body(h_hbm, src_hbm, dst_hbm, ew_hbm, out_hbm,
              src_v, dst_v, ew_v, bb0, bb1, bb2, bb3, fb0, fb1, acc_sh,
              gs0, gs1, gs2, gs3, ss0, ss1):
    cid = lax.axis_index("c")
    sid = lax.axis_index("s")
    base = jnp.where(cid == 0, sid * T0, NS * T0 + sid * T1)
    half_n = pl.multiple_of(jnp.where(cid == 0, T0 // 2, T1 // 2), 8)
    base = pl.multiple_of(base, 8)

    bbufs, gsems = [bb0, bb1, bb2, bb3], [gs0, gs1, gs2, gs3]
    fbufs, ssems = [fb0, fb1], [ss0, ss1]

    def start_gather(buf, gsem, cj):
        pltpu.async_copy(h_hbm.at[src_v.at[cj]], buf, gsem)

    def wait_gather(buf, gsem):
        pltpu.make_async_copy(h_hbm.at[src_v.at[0]], buf, gsem).wait()

    def scale(bbuf, fbuf, cj):
        @pl.loop(0, CHUNK)
        def _(r):
            sv = plsc.load_gather(
                ew_v, [jnp.full((LANES,), cj, jnp.int32),
                       jnp.full((LANES,), r, jnp.int32)])
            for g in range(D // 32):
                w = plsc.bitcast(bbuf[r, pl.ds(g * LANES, LANES)], jnp.bfloat16)
                a, b = plsc.unpack(w, format=plsc.PackFormat.INTERLEAVED)
                fbuf[r, pl.ds(g * 32, LANES)] = a * sv
                fbuf[r, pl.ds(g * 32 + LANES, LANES)] = b * sv

    def start_scat(fbuf, ssem, cj):
        pltpu.async_copy(fbuf, acc_sh.at[dst_v.at[cj]], ssem, add=True)

    def wait_scat(fbuf, ssem):
        pltpu.make_async_copy(fbuf, acc_sh.at[dst_v.at[0]], ssem).wait()

    @pl.loop(0, CHUNK)
    def _(r):
        for jj in range(D // LANES):
            fb0[r, pl.ds(jj * LANES, LANES)] = jnp.zeros((LANES,), jnp.float32)

    for kk in range(ROWBLK // CHUNK):
        off = pl.multiple_of(sid * ROWBLK + kk * CHUNK, 8)
        pltpu.sync_copy(fb0, acc_sh.at[pl.ds(off, CHUNK)])
    plsc.subcore_barrier()

    for half in range(2):
        hb = pl.multiple_of(base + half * half_n, 8)
        pltpu.sync_copy(src_hbm.at[pl.ds(hb, BH)], src_v)
        pltpu.sync_copy(dst_hbm.at[pl.ds(hb, BH)], dst_v)
        pltpu.sync_copy(ew_hbm.at[pl.ds(hb, BH)], ew_v)

        for i in range(4):
            start_gather(bbufs[i], gsems[i], i)
        for i in range(4):
            wait_gather(bbufs[i], gsems[i])
            if i >= 2:
                wait_scat(fbufs[i % 2], ssems[i % 2])
            scale(bbufs[i], fbufs[i % 2], i)
            start_scat(fbufs[i % 2], ssems[i % 2], i)
            start_gather(bbufs[i], gsems[i], jnp.minimum(4 + i, half_n - 1))

        @pl.loop(4, half_n, step=4)
        def _(j):
            for i in range(4):
                wait_gather(bbufs[i], gsems[i])
                wait_scat(fbufs[i % 2], ssems[i % 2])
                scale(bbufs[i], fbufs[i % 2], j + i)
                start_scat(fbufs[i % 2], ssems[i % 2], j + i)
                start_gather(bbufs[i], gsems[i],
                             jnp.minimum(j + i + 4, half_n - 1))

        for i in range(4):
            wait_gather(bbufs[i], gsems[i])
        wait_scat(fb0, ss0)
        wait_scat(fb1, ss1)

    plsc.subcore_barrier()
    for kk in range(ROWBLK // CHUNK):
        off = pl.multiple_of(sid * ROWBLK + kk * CHUNK, 8)
        pltpu.sync_copy(acc_sh.at[pl.ds(off, CHUNK)],
                        out_hbm.at[cid, pl.ds(off, CHUNK)])


def _agg_call(hpb, src2, dst2, ew2):
    k = functools.partial(
        pl.kernel,
        out_type=jax.ShapeDtypeStruct((NC, NPAD, D), jnp.float32),
        mesh=_mesh,
        scratch_types=[
            pltpu.VMEM((BH, CHUNK), jnp.int32),
            pltpu.VMEM((BH, CHUNK), jnp.int32),
            pltpu.VMEM((BH, CHUNK), jnp.float32),
            pltpu.VMEM((CHUNK, D // 2), jnp.int32),
            pltpu.VMEM((CHUNK, D // 2), jnp.int32),
            pltpu.VMEM((CHUNK, D // 2), jnp.int32),
            pltpu.VMEM((CHUNK, D // 2), jnp.int32),
            pltpu.VMEM((CHUNK, D), jnp.float32),
            pltpu.VMEM((CHUNK, D), jnp.float32),
            pltpu.VMEM_SHARED((NPAD, D), jnp.float32),
            pltpu.SemaphoreType.DMA,
            pltpu.SemaphoreType.DMA,
            pltpu.SemaphoreType.DMA,
            pltpu.SemaphoreType.DMA,
            pltpu.SemaphoreType.DMA,
            pltpu.SemaphoreType.DMA,
        ],
        compiler_params=_sc_params,
    )(_agg_body)
    return k(hpb, src2, dst2, ew2)


def _matmul_body(x_ref, w_ref, o_ref):
    o_ref[...] = jnp.dot(x_ref[...], w_ref[...],
                         preferred_element_type=jnp.float32)


def _matmul_call(xp, W):
    return pl.pallas_call(
        _matmul_body,
        out_shape=jax.ShapeDtypeStruct((NPAD, D), jnp.float32),
    )(xp, W)


def _prep_body(part_ref, h_ref, hq_ref, dinvb_ref, hp_ref, hpb_ref):
    deg = part_ref[0] + part_ref[1] + 1.0
    dinv = jnp.where(deg > 0, lax.rsqrt(deg), 0.0)
    dinvb = jnp.broadcast_to(dinv[:, :, None], h_ref.shape)
    dinvb_ref[...] = dinvb
    hp_ref[...] = dinvb * h_ref[...]
    hpb_ref[...] = (dinvb * hq_ref[...]).astype(jnp.bfloat16)


def _prep_call(deg_part3, h1_3, h1q_3):
    r = NPAD // D
    return pl.pallas_call(
        _prep_body,
        out_shape=(jax.ShapeDtypeStruct((r, D, D), jnp.float32),
                   jax.ShapeDtypeStruct((r, D, D), jnp.float32),
                   jax.ShapeDtypeStruct((r, D, D), jnp.bfloat16)),
    )(deg_part3, h1_3, h1q_3)


def _mid_body(part_ref, hp_ref, dinvb_ref, w_ref, wq_ref, b_ref, a_ref,
              o_ref, ob_ref):
    a = a_ref[0, 0]
    z = dinvb_ref[...] * (part_ref[0] + part_ref[1] + hp_ref[...]) + b_ref[...]
    act = jnp.maximum(z, 0.0) + a * jnp.minimum(z, 0.0)
    h2 = jnp.dot(act, w_ref[...], preferred_element_type=jnp.float32)
    h2q = jnp.dot(act, wq_ref[...], preferred_element_type=jnp.float32)
    o_ref[...] = dinvb_ref[...] * h2
    ob_ref[...] = (dinvb_ref[...] * h2q).astype(jnp.bfloat16)


def _mid_call(part1, hp1, dinvb, W2, W2q, b1, prelu_a):
    return pl.pallas_call(
        _mid_body,
        out_shape=(jax.ShapeDtypeStruct((NPAD, D), jnp.float32),
                   jax.ShapeDtypeStruct((NPAD, D), jnp.bfloat16)),
    )(part1, hp1, dinvb, W2, W2q, b1, prelu_a)


def _fin_body(part_ref, hp_ref, dinvb_ref, b_ref, o_ref):
    o_ref[...] = (dinvb_ref[...] * (part_ref[0] + part_ref[1] + hp_ref[...])
                  + b_ref[...])


def _fin_call(part2, hp2, dinvb, b2):
    return pl.pallas_call(
        _fin_body,
        out_shape=jax.ShapeDtypeStruct((NPAD, D), jnp.float32),
    )(part2, hp2, dinvb, b2)


def kernel(x, edge_index, edge_weight, W1, b1, W2, b2, prelu_a):
    n = x.shape[0]
    e = edge_weight.shape[0]

    dept = -(-e // (NW * DCHUNK * 8)) * DCHUNK * 8
    dnch = dept // DCHUNK
    dpad = dept * NW

    assert NS * (T0 + T1) * CHUNK >= e
    apad = (NS * T0 + (NS - 1) * T1 + T1 // 2 + BH) * CHUNK

    pad = max(dpad, apad)
    src_p = jnp.pad(edge_index[0], (0, pad - e))
    dst_p = jnp.pad(edge_index[1], (0, pad - e))
    ew_p = jnp.pad(edge_weight, (0, pad - e))
    xp = jnp.pad(x, ((0, NPAD - n), (0, 0)))

    srcA = src_p[:apad].reshape(apad // CHUNK, CHUNK)
    dstA = dst_p[:apad].reshape(apad // CHUNK, CHUNK)
    ewA = ew_p[:apad].reshape(apad // CHUNK, CHUNK)
    dstD = dst_p[:dpad].reshape(dpad // DCHUNK, DCHUNK)
    ewD = ew_p[:dpad].reshape(dpad // DCHUNK, DCHUNK)

    W1q = W1[:, _P]
    W2q = W2[:, _P]
    deg_part = _deg_call(dstD, ewD, dnch)
    h1 = _matmul_call(xp, W1)
    h1q = _matmul_call(xp, W1q)
    dinvb3, hp1_3, hp1b_3 = _prep_call(
        deg_part.reshape(NC, NPAD // D, D),
        h1.reshape(NPAD // D, D, D),
        h1q.reshape(NPAD // D, D, D))
    dinvb = dinvb3.reshape(NPAD, D)
    hp1 = hp1_3.reshape(NPAD, D)

    def _as_i32(hb):
        return lax.bitcast_convert_type(
            hb.reshape(NPAD, D // 2, 2), jnp.int32)

    hp1b = _as_i32(hp1b_3.reshape(NPAD, D))
    part1 = _agg_call(hp1b, srcA, dstA, ewA)
    hp2, hp2b = _mid_call(part1, hp1, dinvb, W2, W2q,
                          b1.reshape(1, D), prelu_a.reshape(1, 1))
    part2 = _agg_call(_as_i32(hp2b), srcA, dstA, ewA)
    outp = _fin_call(part2, hp2, dinvb, b2.reshape(1, D))
    return outp[:n]
